# baseline (speedup 1.0000x reference)
import functools

import jax
import jax.numpy as jnp
from jax import lax
from jax.experimental import pallas as pl
from jax.experimental.pallas import tpu as pltpu
from jax.experimental.pallas import tpu_sc as plsc

NC = 2
NS = 16
L = 16

CHUNK = 80


def _spmm_body(nchunks, n, dh,
               src_hbm, dst_hbm, val_hbm, x_hbm, out_hbm,
               src_v, dst_v, val_v, bufs, fbufs, zbuf, acc,
               gs0, gs1, gs2, gs3, ss0, ss1):
    gsems = (gs0, gs1, gs2, gs3)
    ssems = (ss0, ss1)
    c = lax.axis_index("c")
    s = lax.axis_index("s")

    pltpu.sync_copy(src_hbm.at[s], src_v)
    pltpu.sync_copy(dst_hbm.at[s], dst_v)
    pltpu.sync_copy(val_hbm.at[s], val_v)

    slab = (n // NS) & ~7
    rem = n - NS * slab
    nd16 = dh // L

    zrows = zbuf.shape[0]

    def zrow(i, carry):
        for q in range(nd16):
            zbuf[i, pl.ds(q * L, L)] = jnp.zeros((L,), jnp.float32)
        return carry

    lax.fori_loop(0, zrows, zrow, 0)
    for k in range(slab // zrows):
        pltpu.sync_copy(zbuf, acc.at[pl.ds(s * slab + k * zrows, zrows)])

    if rem:
        @pl.when(s == NS - 1)
        def _():
            pltpu.sync_copy(zbuf.at[pl.ds(0, rem)], acc.at[pl.ds(NS * slab, rem)])

    plsc.subcore_barrier()

    def gather_start(j, b):
        pltpu.async_copy(x_hbm.at[c].at[src_v.at[j]], bufs.at[b], gsems[b])

    def gather_wait(j, b):
        pltpu.make_async_copy(x_hbm.at[c].at[src_v.at[j]], bufs.at[b],
                              gsems[b]).wait()

    def scat_start(j, fb):
        pltpu.async_copy(fbufs.at[fb], acc.at[dst_v.at[j]], ssems[fb],
                         add=True)

    def scat_wait(j, fb):
        pltpu.make_async_copy(fbufs.at[fb], acc.at[dst_v.at[j]],
                              ssems[fb]).wait()

    def compute(j, b, fb):
        for g in range(CHUNK // L):
            vals = val_v[j, pl.ds(g * L, L)]
            for i in range(L):
                v = lax.gather(
                    vals, jnp.full((L, 1), i, jnp.int32),
                    lax.GatherDimensionNumbers(
                        offset_dims=(), collapsed_slice_dims=(0,),
                        start_index_map=(0,)),
                    (1,), mode=lax.GatherScatterMode.PROMISE_IN_BOUNDS)
                r = g * L + i
                for g2 in range(dh // (2 * L)):
                    xi = bufs[b, r, pl.ds(g2 * L, L)]
                    ev = lax.bitcast_convert_type(
                        jnp.left_shift(xi, 16), jnp.float32)
                    od = lax.bitcast_convert_type(
                        jnp.bitwise_and(xi, jnp.int32(-65536)), jnp.float32)
                    fbufs[fb, r, pl.ds(g2 * 2 * L, L)] = ev * v
                    fbufs[fb, r, pl.ds(g2 * 2 * L + L, L)] = od * v

    def step(j, b, wait_scat):
        gather_wait(j, b)
        fb = b % 2
        if wait_scat:
            scat_wait(j - 2, fb)
        gather_start(j + 2, (b + 2) % 4)
        compute(j, b, fb)
        scat_start(j, fb)

    gather_start(0, 0)
    gather_start(1, 1)
    step(0, 0, False)
    step(1, 1, False)
    step(2, 2, True)
    step(3, 3, True)

    def quad(i, carry):
        for b in range(4):
            step(i * 4 + b, b, True)
        return carry

    lax.fori_loop(1, (nchunks - 2) // 4, quad, 0)

    for j, b in ((nchunks - 2, 0), (nchunks - 1, 1)):
        gather_wait(j, b)
        scat_wait(j - 2, b % 2)
        compute(j, b, b % 2)
        scat_start(j, b % 2)

    scat_wait(nchunks - 2, 0)
    scat_wait(nchunks - 1, 1)
    plsc.subcore_barrier()

    pltpu.sync_copy(acc.at[pl.ds(s * slab, slab)],
                    out_hbm.at[c, pl.ds(s * slab, slab)])
    if rem:
        @pl.when(s == NS - 1)
        def _():
            pltpu.sync_copy(acc.at[pl.ds(NS * slab, rem)],
                            out_hbm.at[c, pl.ds(NS * slab, rem)])


def _linear_body(dh, p_ref, w_ref, b_ref, o_ref):
    o_ref[...] = (
        jnp.dot(p_ref[0], w_ref[:, :dh].T, preferred_element_type=jnp.float32)
        + jnp.dot(p_ref[1], w_ref[:, dh:].T, preferred_element_type=jnp.float32)
        + b_ref[...]
    )


def kernel(edge_index, edge_values, X, W, b):
    n, d = X.shape
    d_out = W.shape[0]
    dh = d // NC
    e = edge_values.shape[0]
    edges_per_tile = e // NS
    nchunks = edges_per_tile // CHUNK
    assert nchunks % 4 == 2 and nchunks >= 6
    slab = (n // NS) & ~7
    zrows = max(k for k in range(1, 105) if slab % k == 0)

    src = edge_index[1].astype(jnp.int32).reshape(NS, nchunks, CHUNK)
    dst = edge_index[0].astype(jnp.int32).reshape(NS, nchunks, CHUNK)
    val = edge_values.astype(jnp.float32).reshape(NS, nchunks, CHUNK)
    xs = lax.bitcast_convert_type(
        jnp.stack([X[:, :dh], X[:, dh:]])
        .astype(jnp.bfloat16).reshape(NC, n, dh // 2, 2),
        jnp.int32)

    perm = []
    for g2 in range(dh // 32):
        perm += [g2 * 32 + 2 * k for k in range(16)]
        perm += [g2 * 32 + 2 * k + 1 for k in range(16)]
    cols = [cc * dh + p for cc in range(NC) for p in perm]
    wp = W[:, jnp.array(cols, dtype=jnp.int32)]

    mesh = plsc.VectorSubcoreMesh(core_axis_name="c", subcore_axis_name="s")
    spmm = pl.kernel(
        functools.partial(_spmm_body, nchunks, n, dh),
        out_type=jax.ShapeDtypeStruct((NC, n, dh), jnp.float32),
        mesh=mesh,
        compiler_params=pltpu.CompilerParams(use_tc_tiling_on_sc=False),
        scratch_types=[
            pltpu.VMEM((nchunks, CHUNK), jnp.int32),
            pltpu.VMEM((nchunks, CHUNK), jnp.int32),
            pltpu.VMEM((nchunks, CHUNK), jnp.float32),
            pltpu.VMEM((4, CHUNK, dh // 2), jnp.int32),
            pltpu.VMEM((2, CHUNK, dh), jnp.float32),
            pltpu.VMEM((zrows, dh), jnp.float32),
            pltpu.VMEM_SHARED((n, dh), jnp.float32),
            pltpu.SemaphoreType.DMA,
            pltpu.SemaphoreType.DMA,
            pltpu.SemaphoreType.DMA,
            pltpu.SemaphoreType.DMA,
            pltpu.SemaphoreType.DMA,
            pltpu.SemaphoreType.DMA,
        ],
    )
    partials = spmm(src, dst, val, xs)

    blk = 1000
    grid = n // blk
    out = pl.pallas_call(
        functools.partial(_linear_body, dh),
        grid=(grid,),
        in_specs=[
            pl.BlockSpec((NC, blk, dh), lambda i: (0, i, 0)),
            pl.BlockSpec((d_out, d), lambda i: (0, 0)),
            pl.BlockSpec((1, d_out), lambda i: (0, 0)),
        ],
        out_specs=pl.BlockSpec((blk, d_out), lambda i: (i, 0)),
        out_shape=jax.ShapeDtypeStruct((n, d_out), jnp.float32),
    )(partials, wp, b.reshape(1, d_out))
    return out


# speedup vs baseline: 1.2460x; 1.2460x over previous
import functools

import jax
import jax.numpy as jnp
from jax import lax
from jax.experimental import pallas as pl
from jax.experimental.pallas import tpu as pltpu
from jax.experimental.pallas import tpu_sc as plsc

NC = 2
NS = 16
L = 16

CHUNK = 80


def _spmm_body(nchunks, n, dh,
               src_hbm, dst_hbm, val_hbm, x_hbm, out_hbm,
               src_v, dst_v, val_v, bufs, zbuf, acc,
               gs0, gs1, gs2, gs3, ss0, ss1, ss2, ss3):
    gsems = (gs0, gs1, gs2, gs3)
    ssems = (ss0, ss1, ss2, ss3)
    c = lax.axis_index("c")
    s = lax.axis_index("s")

    pltpu.sync_copy(src_hbm.at[s], src_v)
    pltpu.sync_copy(dst_hbm.at[s], dst_v)
    pltpu.sync_copy(val_hbm.at[s], val_v)

    slab = (n // NS) & ~7
    rem = n - NS * slab
    nd16 = dh // L

    zrows = zbuf.shape[0]

    def zrow(i, carry):
        for q in range(nd16):
            zbuf[i, pl.ds(q * L, L)] = jnp.zeros((L,), jnp.float32)
        return carry

    lax.fori_loop(0, zrows, zrow, 0)
    for k in range(slab // zrows):
        pltpu.sync_copy(zbuf, acc.at[pl.ds(s * slab + k * zrows, zrows)])

    if rem:
        @pl.when(s == NS - 1)
        def _():
            pltpu.sync_copy(zbuf.at[pl.ds(0, rem)], acc.at[pl.ds(NS * slab, rem)])

    plsc.subcore_barrier()

    def gather_start(j, b):
        pltpu.async_copy(x_hbm.at[c].at[src_v.at[j]], bufs.at[b], gsems[b])

    def gather_wait(j, b):
        pltpu.make_async_copy(x_hbm.at[c].at[src_v.at[j]], bufs.at[b],
                              gsems[b]).wait()

    def scat_start(j, b):
        pltpu.async_copy(bufs.at[b], acc.at[dst_v.at[j]], ssems[b], add=True)

    def scat_wait(j, b):
        pltpu.make_async_copy(bufs.at[b], acc.at[dst_v.at[j]],
                              ssems[b]).wait()

    def compute(j, b):
        for g in range(CHUNK // L):
            vals = val_v[j, pl.ds(g * L, L)]
            for i in range(L):
                v = lax.gather(
                    vals, jnp.full((L, 1), i, jnp.int32),
                    lax.GatherDimensionNumbers(
                        offset_dims=(), collapsed_slice_dims=(0,),
                        start_index_map=(0,)),
                    (1,), mode=lax.GatherScatterMode.PROMISE_IN_BOUNDS)
                r = g * L + i
                for q in range(nd16):
                    bufs[b, r, pl.ds(q * L, L)] = (
                        bufs[b, r, pl.ds(q * L, L)] * v)

    def step(j, b, wait_scat):
        gather_wait(j, b)
        b2 = (b + 2) % 4
        if wait_scat:
            scat_wait(j - 2, b2)
        gather_start(j + 2, b2)
        compute(j, b)
        scat_start(j, b)

    gather_start(0, 0)
    gather_start(1, 1)
    step(0, 0, False)
    step(1, 1, False)
    step(2, 2, True)
    step(3, 3, True)

    def quad(i, carry):
        for b in range(4):
            step(i * 4 + b, b, True)
        return carry

    lax.fori_loop(1, (nchunks - 2) // 4, quad, 0)

    for j, b in ((nchunks - 2, 0), (nchunks - 1, 1)):
        gather_wait(j, b)
        compute(j, b)
        scat_start(j, b)

    for j in range(nchunks - 4, nchunks):
        scat_wait(j, j % 4)
    plsc.subcore_barrier()

    pltpu.sync_copy(acc.at[pl.ds(s * slab, slab)],
                    out_hbm.at[c, pl.ds(s * slab, slab)])
    if rem:
        @pl.when(s == NS - 1)
        def _():
            pltpu.sync_copy(acc.at[pl.ds(NS * slab, rem)],
                            out_hbm.at[c, pl.ds(NS * slab, rem)])


def _linear_body(dh, p_ref, w_ref, b_ref, o_ref):
    o_ref[...] = (
        jnp.dot(p_ref[0], w_ref[:, :dh].T, preferred_element_type=jnp.float32)
        + jnp.dot(p_ref[1], w_ref[:, dh:].T, preferred_element_type=jnp.float32)
        + b_ref[...]
    )


def kernel(edge_index, edge_values, X, W, b):
    n, d = X.shape
    d_out = W.shape[0]
    dh = d // NC
    e = edge_values.shape[0]
    edges_per_tile = e // NS
    nchunks = edges_per_tile // CHUNK
    assert nchunks % 4 == 2 and nchunks >= 6
    slab = (n // NS) & ~7
    zrows = max(k for k in range(1, 105) if slab % k == 0)

    src = edge_index[1].astype(jnp.int32).reshape(NS, nchunks, CHUNK)
    dst = edge_index[0].astype(jnp.int32).reshape(NS, nchunks, CHUNK)
    val = edge_values.astype(jnp.float32).reshape(NS, nchunks, CHUNK)
    xs = jnp.stack([X[:, :dh], X[:, dh:]])

    mesh = plsc.VectorSubcoreMesh(core_axis_name="c", subcore_axis_name="s")
    spmm = pl.kernel(
        functools.partial(_spmm_body, nchunks, n, dh),
        out_type=jax.ShapeDtypeStruct((NC, n, dh), jnp.float32),
        mesh=mesh,
        compiler_params=pltpu.CompilerParams(use_tc_tiling_on_sc=False),
        scratch_types=[
            pltpu.VMEM((nchunks, CHUNK), jnp.int32),
            pltpu.VMEM((nchunks, CHUNK), jnp.int32),
            pltpu.VMEM((nchunks, CHUNK), jnp.float32),
            pltpu.VMEM((4, CHUNK, dh), jnp.float32),
            pltpu.VMEM((zrows, dh), jnp.float32),
            pltpu.VMEM_SHARED((n, dh), jnp.float32),
            pltpu.SemaphoreType.DMA,
            pltpu.SemaphoreType.DMA,
            pltpu.SemaphoreType.DMA,
            pltpu.SemaphoreType.DMA,
            pltpu.SemaphoreType.DMA,
            pltpu.SemaphoreType.DMA,
            pltpu.SemaphoreType.DMA,
            pltpu.SemaphoreType.DMA,
        ],
    )
    partials = spmm(src, dst, val, xs)

    blk = 1000
    grid = n // blk
    out = pl.pallas_call(
        functools.partial(_linear_body, dh),
        grid=(grid,),
        in_specs=[
            pl.BlockSpec((NC, blk, dh), lambda i: (0, i, 0)),
            pl.BlockSpec((d_out, d), lambda i: (0, 0)),
            pl.BlockSpec((1, d_out), lambda i: (0, 0)),
        ],
        out_specs=pl.BlockSpec((blk, d_out), lambda i: (i, 0)),
        out_shape=jax.ShapeDtypeStruct((n, d_out), jnp.float32),
    )(partials, W, b.reshape(1, d_out))
    return out
